# Initial kernel scaffold; baseline (speedup 1.0000x reference)
import jax
import jax.numpy as jnp
from jax.experimental import pallas as pl


def kernel(x):
    raise NotImplementedError("write your pallas kernel here")


# speedup vs baseline: 3.7310x; 3.7310x over previous
import jax
import jax.numpy as jnp
from jax.experimental import pallas as pl
from jax.experimental.pallas import tpu as pltpu

_CONF = 0.25
_IOU = 0.45
_NMS_N = 1000
_PAD_N = 1024
_DET = 300
_DET_PAD = 304
_MAX_WH = 4096.0
_PREP_BLK = 500
_IOU_BLK = 32


def _prep_kernel(x_ref, score_ref, cls_ref, box_ref):
    nblk = x_ref.shape[1] // _PREP_BLK

    def blk(bi, _):
        r = pl.ds(bi * _PREP_BLK, _PREP_BLK)
        xb = x_ref[0, r, :]
        obj = xb[:, 4:5]
        cc = xb[:, 5:] * obj
        conf = jnp.max(cc, axis=1, keepdims=True)
        cio = jax.lax.broadcasted_iota(jnp.int32, cc.shape, 1).astype(jnp.float32)
        cls = jnp.min(jnp.where(cc == conf, cio, 1e9), axis=1, keepdims=True)
        sc = jnp.where(conf > _CONF, conf, -1.0)
        xy = xb[:, 0:2]
        wh = xb[:, 2:4]
        box = jnp.concatenate([xy - wh / 2.0, xy + wh / 2.0], axis=1)
        score_ref[0, r, :] = sc
        cls_ref[0, r, :] = cls
        box_ref[0, r, :] = box
        return 0

    jax.lax.fori_loop(0, nblk, blk, 0)


def _nms_kernel(data_ref, dataT_ref, out_ref, a_ref):
    dt = dataT_ref[0]
    offr = dt[5:6, :] * _MAX_WH
    x1r = dt[0:1, :] + offr
    y1r = dt[1:2, :] + offr
    x2r = dt[2:3, :] + offr
    y2r = dt[3:4, :] + offr
    arear = (x2r - x1r) * (y2r - y1r)
    scr = dt[4:5, :]

    def iou_blk(bi, _):
        r = pl.ds(bi * _IOU_BLK, _IOU_BLK)
        db = data_ref[0, r, :]
        off = db[:, 5:6] * _MAX_WH
        x1 = db[:, 0:1] + off
        y1 = db[:, 1:2] + off
        x2 = db[:, 2:3] + off
        y2 = db[:, 3:4] + off
        area = (x2 - x1) * (y2 - y1)
        w = jnp.clip(jnp.minimum(x2, x2r) - jnp.maximum(x1, x1r), 0.0, None)
        h = jnp.clip(jnp.minimum(y2, y2r) - jnp.maximum(y1, y1r), 0.0, None)
        inter = w * h
        iou = inter / (area + arear - inter + 1e-9)
        a_ref[r, :] = (iou > _IOU).astype(jnp.float32)
        return 0

    jax.lax.fori_loop(0, _PAD_N // _IOU_BLK, iou_blk, 0)

    lane = jax.lax.broadcasted_iota(jnp.int32, (1, _PAD_N), 1)
    keep0 = (scr > _CONF).astype(jnp.float32)

    def body(i, keep):
        row = a_ref[pl.ds(i, 1), :]
        masked = row * keep * (lane < i).astype(jnp.float32)
        supp = jnp.sum(masked)
        flag = jnp.where(supp > 0.0, 0.0, 1.0)
        return jnp.where(lane == i, keep * flag, keep)

    keep = jax.lax.fori_loop(0, _NMS_N, body, keep0)

    inc = keep
    sh = 1
    while sh < _PAD_N:
        rolled = jnp.concatenate([inc[:, -sh:], inc[:, :-sh]], axis=1)
        inc = inc + jnp.where(lane >= sh, rolled, 0.0)
        sh *= 2
    pos = inc - keep

    d = data_ref[0]
    half = _DET_PAD // 2
    srow = jax.lax.broadcasted_iota(jnp.int32, (half, 1), 0).astype(jnp.float32)
    for h in range(2):
        svals = srow + float(h * half)
        m = jnp.where((pos == svals) & (keep > 0.0), 1.0, 0.0)
        det = jnp.dot(m, d, preferred_element_type=jnp.float32)
        out_ref[0, pl.ds(h * half, half), :] = det


def kernel(x):
    xs = x[0]
    nimg, n, _ = xs.shape

    chunk = 2000
    score, cls, box = pl.pallas_call(
        _prep_kernel,
        grid=(nimg, n // chunk),
        in_specs=[pl.BlockSpec((1, chunk, 85), lambda b, c: (b, c, 0))],
        out_specs=[
            pl.BlockSpec((1, chunk, 1), lambda b, c: (b, c, 0)),
            pl.BlockSpec((1, chunk, 1), lambda b, c: (b, c, 0)),
            pl.BlockSpec((1, chunk, 4), lambda b, c: (b, c, 0)),
        ],
        out_shape=[
            jax.ShapeDtypeStruct((nimg, n, 1), jnp.float32),
            jax.ShapeDtypeStruct((nimg, n, 1), jnp.float32),
            jax.ShapeDtypeStruct((nimg, n, 4), jnp.float32),
        ],
    )(xs)

    scores = score[..., 0]
    sc_top, idx = jax.lax.top_k(scores, _NMS_N)
    b = jnp.take_along_axis(box, idx[..., None], axis=1)
    c = jnp.take_along_axis(cls[..., 0], idx, axis=1)
    data = jnp.concatenate(
        [b, sc_top[..., None], c[..., None],
         jnp.zeros((nimg, _NMS_N, 2), jnp.float32)], axis=-1)
    data = jnp.pad(data, ((0, 0), (0, _PAD_N - _NMS_N), (0, 0)))
    dataT = data.transpose(0, 2, 1)

    out = pl.pallas_call(
        _nms_kernel,
        grid=(nimg,),
        in_specs=[
            pl.BlockSpec((1, _PAD_N, 8), lambda b: (b, 0, 0)),
            pl.BlockSpec((1, 8, _PAD_N), lambda b: (b, 0, 0)),
        ],
        out_specs=pl.BlockSpec((1, _DET_PAD, 8), lambda b: (b, 0, 0)),
        out_shape=jax.ShapeDtypeStruct((nimg, _DET_PAD, 8), jnp.float32),
        scratch_shapes=[pltpu.VMEM((_PAD_N, _PAD_N), jnp.float32)],
    )(data, dataT)

    return out[:, :_DET, :6]


# speedup vs baseline: 6.6469x; 1.7815x over previous
import jax
import jax.numpy as jnp
from jax.experimental import pallas as pl
from jax.experimental.pallas import tpu as pltpu

_CONF = 0.25
_IOU = 0.45
_NMS_N = 1000
_PAD_N = 1024
_DET = 300
_DET_PAD = 304
_MAX_WH = 4096.0
_PREP_BLK = 500
_IOU_BLK = 32


def _prep_kernel(x_ref, score_ref, cls_ref, box_ref):
    nblk = x_ref.shape[1] // _PREP_BLK

    def blk(bi, _):
        r = pl.ds(bi * _PREP_BLK, _PREP_BLK)
        xb = x_ref[0, r, :]
        obj = xb[:, 4:5]
        cc = xb[:, 5:] * obj
        conf = jnp.max(cc, axis=1, keepdims=True)
        cio = jax.lax.broadcasted_iota(jnp.int32, cc.shape, 1).astype(jnp.float32)
        cls = jnp.min(jnp.where(cc == conf, cio, 1e9), axis=1, keepdims=True)
        sc = jnp.where(conf > _CONF, conf, -1.0)
        xy = xb[:, 0:2]
        wh = xb[:, 2:4]
        box = jnp.concatenate([xy - wh / 2.0, xy + wh / 2.0], axis=1)
        score_ref[0, r, :] = sc
        cls_ref[0, r, :] = cls
        box_ref[0, r, :] = box
        return 0

    jax.lax.fori_loop(0, nblk, blk, 0)


def _nms_kernel(data_ref, dataT_ref, sc4_ref, out_ref, a_ref):
    nimg = data_ref.shape[0]
    for g in range(nimg):
        dt = dataT_ref[g]
        offr = dt[5:6, :] * _MAX_WH
        x1r = dt[0:1, :] + offr
        y1r = dt[1:2, :] + offr
        x2r = dt[2:3, :] + offr
        y2r = dt[3:4, :] + offr
        arear = (x2r - x1r) * (y2r - y1r)

        def iou_blk(bi, _):
            r = pl.ds(bi * _IOU_BLK, _IOU_BLK)
            db = data_ref[g, r, :]
            off = db[:, 5:6] * _MAX_WH
            x1 = db[:, 0:1] + off
            y1 = db[:, 1:2] + off
            x2 = db[:, 2:3] + off
            y2 = db[:, 3:4] + off
            area = (x2 - x1) * (y2 - y1)
            w = jnp.clip(jnp.minimum(x2, x2r) - jnp.maximum(x1, x1r), 0.0, None)
            h = jnp.clip(jnp.minimum(y2, y2r) - jnp.maximum(y1, y1r), 0.0, None)
            inter = w * h
            iou = inter / (area + arear - inter + 1e-9)
            a_ref[r, g, :] = (iou > _IOU).astype(jnp.float32)
            return 0

        jax.lax.fori_loop(0, _PAD_N // _IOU_BLK, iou_blk, 0)

    lane = jax.lax.broadcasted_iota(jnp.int32, (nimg, _PAD_N), 1)
    keep0 = (sc4_ref[...] > _CONF).astype(jnp.float32)

    def body(i, keep):
        row = a_ref[pl.ds(i, 1)][0]
        masked = row * keep * (lane < i).astype(jnp.float32)
        supp = jnp.sum(masked, axis=1, keepdims=True)
        flag = jnp.where(supp > 0.0, 0.0, 1.0)
        return jnp.where(lane == i, keep * flag, keep)

    keep = jax.lax.fori_loop(0, _NMS_N, body, keep0)

    inc = keep
    sh = 1
    while sh < _PAD_N:
        rolled = jnp.concatenate([inc[:, -sh:], inc[:, :-sh]], axis=1)
        inc = inc + jnp.where(lane >= sh, rolled, 0.0)
        sh *= 2
    pos = inc - keep

    half = _DET_PAD // 2
    srow = jax.lax.broadcasted_iota(jnp.int32, (half, 1), 0).astype(jnp.float32)
    for g in range(nimg):
        d = data_ref[g]
        pos_g = pos[g:g + 1, :]
        keep_g = keep[g:g + 1, :]
        for h in range(2):
            svals = srow + float(h * half)
            m = jnp.where((pos_g == svals) & (keep_g > 0.0), 1.0, 0.0)
            det = jnp.dot(m, d, preferred_element_type=jnp.float32)
            out_ref[g, pl.ds(h * half, half), :] = det


def kernel(x):
    xs = x[0]
    nimg, n, _ = xs.shape

    chunk = 2000
    score, cls, box = pl.pallas_call(
        _prep_kernel,
        grid=(nimg, n // chunk),
        in_specs=[pl.BlockSpec((1, chunk, 85), lambda b, c: (b, c, 0))],
        out_specs=[
            pl.BlockSpec((1, chunk, 1), lambda b, c: (b, c, 0)),
            pl.BlockSpec((1, chunk, 1), lambda b, c: (b, c, 0)),
            pl.BlockSpec((1, chunk, 4), lambda b, c: (b, c, 0)),
        ],
        out_shape=[
            jax.ShapeDtypeStruct((nimg, n, 1), jnp.float32),
            jax.ShapeDtypeStruct((nimg, n, 1), jnp.float32),
            jax.ShapeDtypeStruct((nimg, n, 4), jnp.float32),
        ],
    )(xs)

    scores = score[..., 0]
    sc_top, idx = jax.lax.top_k(scores, _NMS_N)
    b = jnp.take_along_axis(box, idx[..., None], axis=1)
    c = jnp.take_along_axis(cls[..., 0], idx, axis=1)
    data = jnp.concatenate(
        [b, sc_top[..., None], c[..., None],
         jnp.zeros((nimg, _NMS_N, 2), jnp.float32)], axis=-1)
    data = jnp.pad(data, ((0, 0), (0, _PAD_N - _NMS_N), (0, 0)))
    dataT = data.transpose(0, 2, 1)
    sc4 = jnp.pad(sc_top, ((0, 0), (0, _PAD_N - _NMS_N)))

    out = pl.pallas_call(
        _nms_kernel,
        out_shape=jax.ShapeDtypeStruct((nimg, _DET_PAD, 8), jnp.float32),
        scratch_shapes=[pltpu.VMEM((_PAD_N, nimg, _PAD_N), jnp.float32)],
    )(data, dataT, sc4)

    return out[:, :_DET, :6]


# speedup vs baseline: 7.8152x; 1.1758x over previous
import jax
import jax.numpy as jnp
from jax.experimental import pallas as pl
from jax.experimental.pallas import tpu as pltpu

_CONF = 0.25
_IOU = 0.45
_NMS_N = 1000
_PAD_N = 1024
_DET = 300
_DET_PAD = 304
_MAX_WH = 4096.0
_PREP_BLK = 500
_IOU_BLK = 32


def _prep_kernel(x_ref, score_ref, cls_ref, box_ref):
    nblk = x_ref.shape[2] // _PREP_BLK

    def blk(bi, _):
        r = pl.ds(bi * _PREP_BLK, _PREP_BLK)
        xb = x_ref[0, 0, r, :]
        obj = xb[:, 4:5]
        cc = xb[:, 5:] * obj
        conf = jnp.max(cc, axis=1, keepdims=True)
        cio = jax.lax.broadcasted_iota(jnp.int32, cc.shape, 1).astype(jnp.float32)
        cls = jnp.min(jnp.where(cc == conf, cio, 1e9), axis=1, keepdims=True)
        sc = jnp.where(conf > _CONF, conf, -1.0)
        xy = xb[:, 0:2]
        wh = xb[:, 2:4]
        box = jnp.concatenate([xy - wh / 2.0, xy + wh / 2.0], axis=1)
        score_ref[0, r, :] = sc
        cls_ref[0, r, :] = cls
        box_ref[0, r, :] = box
        return 0

    jax.lax.fori_loop(0, nblk, blk, 0)


def _nms_kernel(data_ref, out_ref, a_ref, dts_ref):
    nimg = data_ref.shape[0]
    for chunk in range(_PAD_N // 128):
        rows = jax.lax.broadcasted_iota(jnp.int32, (_PAD_N, 128), 0)
        cols = jax.lax.broadcasted_iota(jnp.int32, (_PAD_N, 128), 1) + chunk * 128
        ic = (rows == cols).astype(jnp.float32)
        for g in range(nimg):
            dt_c = jax.lax.dot_general(
                data_ref[g], ic, (((0,), (0,)), ((), ())),
                preferred_element_type=jnp.float32)
            dts_ref[g, :, pl.ds(chunk * 128, 128)] = dt_c

    for g in range(nimg):
        dt = dts_ref[g]
        offr = dt[5:6, :] * _MAX_WH
        x1r = dt[0:1, :] + offr
        y1r = dt[1:2, :] + offr
        x2r = dt[2:3, :] + offr
        y2r = dt[3:4, :] + offr
        arear = (x2r - x1r) * (y2r - y1r)

        def iou_blk(bi, _):
            r = pl.ds(bi * _IOU_BLK, _IOU_BLK)
            db = data_ref[g, r, :]
            off = db[:, 5:6] * _MAX_WH
            x1 = db[:, 0:1] + off
            y1 = db[:, 1:2] + off
            x2 = db[:, 2:3] + off
            y2 = db[:, 3:4] + off
            area = (x2 - x1) * (y2 - y1)
            w = jnp.clip(jnp.minimum(x2, x2r) - jnp.maximum(x1, x1r), 0.0, None)
            h = jnp.clip(jnp.minimum(y2, y2r) - jnp.maximum(y1, y1r), 0.0, None)
            inter = w * h
            iou = inter / (area + arear - inter + 1e-9)
            a_ref[r, g, :] = (iou > _IOU).astype(jnp.float32)
            return 0

        jax.lax.fori_loop(0, _PAD_N // _IOU_BLK, iou_blk, 0)

    lane = jax.lax.broadcasted_iota(jnp.int32, (nimg, _PAD_N), 1)
    scr4 = jnp.concatenate([dts_ref[g, 4:5, :] for g in range(nimg)], axis=0)
    keep0 = (scr4 > _CONF).astype(jnp.float32)

    def body(i, keep):
        row = a_ref[pl.ds(i, 1)][0]
        masked = row * keep * (lane < i).astype(jnp.float32)
        supp = jnp.sum(masked, axis=1, keepdims=True)
        flag = jnp.where(supp > 0.0, 0.0, 1.0)
        return jnp.where(lane == i, keep * flag, keep)

    keep = jax.lax.fori_loop(0, _NMS_N, body, keep0)

    inc = keep
    sh = 1
    while sh < _PAD_N:
        rolled = jnp.concatenate([inc[:, -sh:], inc[:, :-sh]], axis=1)
        inc = inc + jnp.where(lane >= sh, rolled, 0.0)
        sh *= 2
    pos = inc - keep

    half = _DET_PAD // 2
    srow = jax.lax.broadcasted_iota(jnp.int32, (half, 1), 0).astype(jnp.float32)
    for g in range(nimg):
        d = data_ref[g]
        pos_g = pos[g:g + 1, :]
        keep_g = keep[g:g + 1, :]
        for h in range(2):
            svals = srow + float(h * half)
            m = jnp.where((pos_g == svals) & (keep_g > 0.0), 1.0, 0.0)
            det = jnp.dot(m, d, preferred_element_type=jnp.float32)
            out_ref[g, pl.ds(h * half, half), :] = det


def kernel(x):
    _, nimg, n, _ = x.shape

    chunk = 2000
    score, cls, box = pl.pallas_call(
        _prep_kernel,
        grid=(nimg, n // chunk),
        in_specs=[pl.BlockSpec((1, 1, chunk, 85), lambda b, c: (0, b, c, 0))],
        out_specs=[
            pl.BlockSpec((1, chunk, 1), lambda b, c: (b, c, 0)),
            pl.BlockSpec((1, chunk, 1), lambda b, c: (b, c, 0)),
            pl.BlockSpec((1, chunk, 4), lambda b, c: (b, c, 0)),
        ],
        out_shape=[
            jax.ShapeDtypeStruct((nimg, n, 1), jnp.float32),
            jax.ShapeDtypeStruct((nimg, n, 1), jnp.float32),
            jax.ShapeDtypeStruct((nimg, n, 4), jnp.float32),
        ],
    )(x)

    scores = score[..., 0]
    sc_top, idx = jax.lax.top_k(scores, _NMS_N)
    b = jnp.take_along_axis(box, idx[..., None], axis=1)
    c = jnp.take_along_axis(cls[..., 0], idx, axis=1)
    data = jnp.concatenate(
        [b, sc_top[..., None], c[..., None],
         jnp.zeros((nimg, _NMS_N, 2), jnp.float32)], axis=-1)
    data = jnp.pad(data, ((0, 0), (0, _PAD_N - _NMS_N), (0, 0)))

    out = pl.pallas_call(
        _nms_kernel,
        out_shape=jax.ShapeDtypeStruct((nimg, _DET_PAD, 8), jnp.float32),
        scratch_shapes=[
            pltpu.VMEM((_PAD_N, nimg, _PAD_N), jnp.float32),
            pltpu.VMEM((nimg, 8, _PAD_N), jnp.float32),
        ],
    )(data)

    return out[:, :_DET, :6]
